# baseline (speedup 1.0000x reference)
import jax
import jax.numpy as jnp
from jax import lax
from jax.experimental import pallas as pl
from jax.experimental.pallas import tpu as pltpu
from jax.experimental.pallas import tpu_sc as plsc

NC = 2
NS = 16
L = 16
NW = NC * NS

B = 16384
D = 64
BPW = B // NW
CHUNK = 128
NCH = BPW // CHUNK
GROUPS = BPW // L


def _sc_body(wc_hbm, wo_hbm, emb1_hbm, emb2_hbm, out_hbm,
             idx1_v, idx2_v, rows1_v, rows2_v, out_v, sem):
    wid = lax.axis_index("s") * NC + lax.axis_index("c")
    base_chunk = wid * NCH

    pltpu.sync_copy(wc_hbm.at[pl.ds(base_chunk, NCH)], idx1_v)
    pltpu.sync_copy(wo_hbm.at[pl.ds(base_chunk, NCH)], idx2_v)

    copies = []
    for j in range(NCH):
        copies.append(pltpu.async_copy(emb1_hbm.at[idx1_v.at[j]],
                                       rows1_v.at[j], sem))
        copies.append(pltpu.async_copy(emb2_hbm.at[idx2_v.at[j]],
                                       rows2_v.at[j], sem))
    for c in copies:
        c.wait()

    lane = lax.iota(jnp.int32, L)

    def group_body(t, _):
        j = t // (CHUNK // L)
        rid = (t % (CHUNK // L)) * L + lane
        jj = jnp.full((L,), j, jnp.int32)
        acc = jnp.zeros((L,), jnp.float32)
        for d in range(D):
            dd = jnp.full((L,), d, jnp.int32)
            v1 = plsc.load_gather(rows1_v, [jj, rid, dd])
            v2 = plsc.load_gather(rows2_v, [jj, rid, dd])
            acc = acc + v1 * v2
        y = 1.0 / (1.0 + jnp.exp(-acc))
        out_v[pl.ds(t * L, L)] = y
        return 0

    lax.fori_loop(0, GROUPS, group_body, 0)

    pltpu.sync_copy(out_v, out_hbm.at[pl.ds(wid * BPW, BPW)])


@jax.jit
def _sc_call(wc, wo, emb1, emb2):
    mesh = plsc.VectorSubcoreMesh(core_axis_name="c", subcore_axis_name="s",
                                  num_cores=NC, num_subcores=NS)
    return pl.kernel(
        _sc_body,
        out_type=jax.ShapeDtypeStruct((B,), jnp.float32),
        mesh=mesh,
        scratch_types=[
            pltpu.VMEM((NCH, CHUNK), jnp.int32),
            pltpu.VMEM((NCH, CHUNK), jnp.int32),
            pltpu.VMEM((NCH, CHUNK, D), jnp.float32),
            pltpu.VMEM((NCH, CHUNK, D), jnp.float32),
            pltpu.VMEM((BPW,), jnp.float32),
            pltpu.SemaphoreType.DMA,
        ],
        compiler_params=pltpu.CompilerParams(needs_layout_passes=False,
                                             use_tc_tiling_on_sc=False),
    )(wc, wo, emb1, emb2)


def kernel(X, emb1, emb2):
    wc = X[:, 0].reshape(B // CHUNK, CHUNK)
    wo = X[:, 1].reshape(B // CHUNK, CHUNK)
    return _sc_call(wc, wo, emb1, emb2)
